# baseline (speedup 1.0000x reference)
import functools

import jax
import jax.numpy as jnp
from jax import lax
from jax.experimental import pallas as pl
from jax.experimental.pallas import tpu as pltpu
from jax.experimental.pallas import tpu_sc as plsc

NUM_TOKENS = 32768
N_EXP = 64
NC = 2
NS = 16
NW = NC * NS

SPLIT = 7168
TC_BLOCK = 1024

_KS0 = 0
_KS1 = 42
_KS2 = _KS1 ^ 0x1BD11BDA
_ROT_A = (13, 15, 26, 6)
_ROT_B = (17, 29, 16, 24)


def _rotl(x, r):
    return lax.shift_left(x, jnp.int32(r)) | lax.shift_right_logical(
        x, jnp.int32(32 - r))


def _threefry_bits23(j):
    x1 = j + jnp.int32(_KS1)
    x0 = x1
    x1 = _rotl(x1, _ROT_A[0]) ^ x0
    ks = (jnp.int32(_KS0), jnp.int32(_KS1), jnp.int32(_KS2))
    first = True
    for g in range(5):
        rots = _ROT_A if g % 2 == 0 else _ROT_B
        for r in (rots[1:] if first else rots):
            x0 = x0 + x1
            x1 = _rotl(x1, r) ^ x0
        first = False
        x0 = x0 + ks[(g + 1) % 3]
        x1 = x1 + ks[(g + 2) % 3] + jnp.int32(g + 1)
    return lax.shift_right_logical(x0 ^ x1, jnp.int32(9))


SC_ROWS = SPLIT
SC_ROWS_PER_W = SC_ROWS // NW
SC_GROUPS = SC_ROWS_PER_W // 16
_UNROLL = 8


def _sc_body(out_hbm, buf, bits):
    wid = lax.axis_index("s") * NC + lax.axis_index("c")
    row0 = wid * SC_ROWS_PER_W
    lane = lax.iota(jnp.int32, 16)
    lane64 = lane * jnp.int32(N_EXP)
    neg1 = jnp.full((16,), -1, jnp.int32)
    one = jnp.float32(1.0)
    zero = jnp.float32(0.0)

    def group(g, carry):
        jbase = (row0 + 16 * g) * N_EXP
        row_idx = g * 16 + lane

        def pass1(i, t):
            t63, t62, t61 = t
            e0 = i * _UNROLL
            bs = [_threefry_bits23(lane64 + (jbase + e0 + k))
                  for k in range(_UNROLL)]
            for k, b in enumerate(bs):
                bits[pl.ds((e0 + k) * 16, 16)] = b
                hi = jnp.maximum(t63, b)
                c1 = jnp.minimum(t63, b)
                mid = jnp.maximum(t62, c1)
                c2 = jnp.minimum(t62, c1)
                lo = jnp.maximum(t61, c2)
                t63, t62, t61 = hi, mid, lo
            return t63, t62, t61

        _, t62, t61 = lax.fori_loop(0, N_EXP // _UNROLL, pass1,
                                    (neg1, neg1, neg1))
        strict = t62 > t61

        def pass2(i, carry2):
            e0 = i * _UNROLL
            for k in range(_UNROLL):
                b = bits[pl.ds((e0 + k) * 16, 16)]
                sel = (b > t62) | ((b == t62) & strict)
                col = jnp.zeros((16,), jnp.int32) + jnp.int32(e0 + k)
                plsc.store_scatter(buf, [row_idx, col],
                                   jnp.where(sel, one, zero))
            return carry2

        lax.fori_loop(0, N_EXP // _UNROLL, pass2, 0)
        return carry

    lax.fori_loop(0, SC_GROUPS, group, 0)
    pltpu.sync_copy(buf, out_hbm.at[pl.ds(row0, SC_ROWS_PER_W)])


@functools.cache
def _sc_gate_fn():
    return pl.kernel(
        _sc_body,
        out_type=jax.ShapeDtypeStruct((SC_ROWS, N_EXP), jnp.float32),
        mesh=plsc.VectorSubcoreMesh(core_axis_name="c", subcore_axis_name="s"),
        scratch_types=[
            pltpu.VMEM((SC_ROWS_PER_W, N_EXP), jnp.float32),
            pltpu.VMEM((N_EXP * 16,), jnp.int32),
        ],
        compiler_params=pltpu.CompilerParams(needs_layout_passes=False),
    )


TC_ROWS = NUM_TOKENS - SPLIT


def _tc_kernel(out_ref):
    i = pl.program_id(0)
    row0 = jnp.int32(SPLIT) + i * jnp.int32(TC_BLOCK)
    j = (row0 * jnp.int32(N_EXP)
         + lax.broadcasted_iota(jnp.int32, (N_EXP, TC_BLOCK), 1)
         * jnp.int32(N_EXP)
         + lax.broadcasted_iota(jnp.int32, (N_EXP, TC_BLOCK), 0))
    b = _threefry_bits23(j)
    m1 = jnp.max(b, axis=0, keepdims=True)
    e1 = b == m1
    z = jnp.where(e1, jnp.int32(-1), b)
    m2 = jnp.max(z, axis=0, keepdims=True)
    e2 = z == m2
    y = jnp.where(e2, jnp.int32(-1), z)
    m3 = jnp.max(y, axis=0, keepdims=True)
    c1 = jnp.sum(e1.astype(jnp.int32), axis=0, keepdims=True)
    c2 = jnp.sum(e2.astype(jnp.int32), axis=0, keepdims=True)
    t62 = jnp.where(c1 >= 2, m1, m2)
    t61 = jnp.where(c1 >= 3, m1, jnp.where(c1 + c2 >= 3, m2, m3))
    sel = (b > t62) | ((b == t62) & (t62 > t61))
    mask = jnp.where(sel, jnp.float32(1.0), jnp.float32(0.0))
    out_ref[...] = mask.T


_tc_gate = pl.pallas_call(
    _tc_kernel,
    out_shape=jax.ShapeDtypeStruct((NUM_TOKENS, N_EXP), jnp.float32),
    grid=(TC_ROWS // TC_BLOCK,),
    out_specs=pl.BlockSpec((TC_BLOCK, N_EXP),
                           lambda i: (i + SPLIT // TC_BLOCK, 0)),
    compiler_params=pltpu.CompilerParams(
        dimension_semantics=("arbitrary",)),
)


def kernel(input):
    assert input.shape[0] == NUM_TOKENS
    full = _tc_gate()
    sc = _sc_gate_fn()()
    return lax.dynamic_update_slice(full, sc, (0, 0))


# speedup vs baseline: 1.0011x; 1.0011x over previous
import functools

import jax
import jax.numpy as jnp
from jax import lax
from jax.experimental import pallas as pl
from jax.experimental.pallas import tpu as pltpu
from jax.experimental.pallas import tpu_sc as plsc

NUM_TOKENS = 32768
N_EXP = 64
NC = 2
NS = 16
NW = NC * NS

SPLIT = 7168
TC_BLOCK = 1024

_KS0 = 0
_KS1 = 42
_KS2 = _KS1 ^ 0x1BD11BDA
_ROT_A = (13, 15, 26, 6)
_ROT_B = (17, 29, 16, 24)


def _rotl(x, r):
    return lax.shift_left(x, jnp.int32(r)) | lax.shift_right_logical(
        x, jnp.int32(32 - r))


def _threefry_bits23(j):
    x1 = j + jnp.int32(_KS1)
    x0 = x1
    x1 = _rotl(x1, _ROT_A[0]) ^ x0
    ks = (jnp.int32(_KS0), jnp.int32(_KS1), jnp.int32(_KS2))
    first = True
    for g in range(5):
        rots = _ROT_A if g % 2 == 0 else _ROT_B
        for r in (rots[1:] if first else rots):
            x0 = x0 + x1
            x1 = _rotl(x1, r) ^ x0
        first = False
        x0 = x0 + ks[(g + 1) % 3]
        x1 = x1 + ks[(g + 2) % 3] + jnp.int32(g + 1)
    return lax.shift_right_logical(x0 ^ x1, jnp.int32(9))


SC_ROWS = SPLIT
SC_ROWS_PER_W = SC_ROWS // NW
SC_GROUPS = SC_ROWS_PER_W // 16
_UNROLL = 8


def _sc_body(out_hbm, buf, bits):
    wid = lax.axis_index("s") * NC + lax.axis_index("c")
    row0 = wid * SC_ROWS_PER_W
    lane = lax.iota(jnp.int32, 16)
    lane64 = lane * jnp.int32(N_EXP)
    neg1 = jnp.full((16,), -1, jnp.int32)
    one = jnp.float32(1.0)
    zero = jnp.float32(0.0)

    def group(g, carry):
        jbase = (row0 + 16 * g) * N_EXP
        row_idx = g * 16 + lane

        def pass1(i, t):
            t63, t62, t61 = t
            e0 = i * _UNROLL
            bs = [_threefry_bits23(lane64 + (jbase + e0 + k))
                  for k in range(_UNROLL)]
            for k, b in enumerate(bs):
                bits[pl.ds((e0 + k) * 16, 16)] = b
                hi = jnp.maximum(t63, b)
                c1 = jnp.minimum(t63, b)
                mid = jnp.maximum(t62, c1)
                c2 = jnp.minimum(t62, c1)
                lo = jnp.maximum(t61, c2)
                t63, t62, t61 = hi, mid, lo
            return t63, t62, t61

        _, t62, t61 = lax.fori_loop(0, N_EXP // _UNROLL, pass1,
                                    (neg1, neg1, neg1))
        strict = t62 > t61

        def pass2(i, carry2):
            e0 = i * _UNROLL
            for k in range(_UNROLL):
                b = bits[pl.ds((e0 + k) * 16, 16)]
                sel = (b > t62) | ((b == t62) & strict)
                col = jnp.zeros((16,), jnp.int32) + jnp.int32(e0 + k)
                plsc.store_scatter(buf, [row_idx, col],
                                   jnp.where(sel, one, zero))
            return carry2

        lax.fori_loop(0, N_EXP // _UNROLL, pass2, 0)
        return carry

    lax.fori_loop(0, SC_GROUPS, group, 0)
    pltpu.sync_copy(buf, out_hbm.at[pl.ds(row0, SC_ROWS_PER_W)])


@functools.cache
def _sc_gate_fn():
    return pl.kernel(
        _sc_body,
        out_type=jax.ShapeDtypeStruct((SC_ROWS, N_EXP), jnp.float32),
        mesh=plsc.VectorSubcoreMesh(core_axis_name="c", subcore_axis_name="s"),
        scratch_types=[
            pltpu.VMEM((SC_ROWS_PER_W, N_EXP), jnp.float32),
            pltpu.VMEM((N_EXP * 16,), jnp.int32),
        ],
        compiler_params=pltpu.CompilerParams(needs_layout_passes=False,
                                             use_tc_tiling_on_sc=True),
    )


TC_ROWS = NUM_TOKENS - SPLIT


def _tc_kernel(out_ref):
    i = pl.program_id(0)
    row0 = jnp.int32(SPLIT) + i * jnp.int32(TC_BLOCK)
    j = (row0 * jnp.int32(N_EXP)
         + lax.broadcasted_iota(jnp.int32, (N_EXP, TC_BLOCK), 1)
         * jnp.int32(N_EXP)
         + lax.broadcasted_iota(jnp.int32, (N_EXP, TC_BLOCK), 0))
    b = _threefry_bits23(j)
    m1 = jnp.max(b, axis=0, keepdims=True)
    e1 = b == m1
    z = jnp.where(e1, jnp.int32(-1), b)
    m2 = jnp.max(z, axis=0, keepdims=True)
    e2 = z == m2
    y = jnp.where(e2, jnp.int32(-1), z)
    m3 = jnp.max(y, axis=0, keepdims=True)
    c1 = jnp.sum(e1.astype(jnp.int32), axis=0, keepdims=True)
    c2 = jnp.sum(e2.astype(jnp.int32), axis=0, keepdims=True)
    t62 = jnp.where(c1 >= 2, m1, m2)
    t61 = jnp.where(c1 >= 3, m1, jnp.where(c1 + c2 >= 3, m2, m3))
    sel = (b > t62) | ((b == t62) & (t62 > t61))
    mask = jnp.where(sel, jnp.float32(1.0), jnp.float32(0.0))
    out_ref[...] = mask.T


_tc_gate = pl.pallas_call(
    _tc_kernel,
    out_shape=jax.ShapeDtypeStruct((NUM_TOKENS, N_EXP), jnp.float32),
    grid=(TC_ROWS // TC_BLOCK,),
    out_specs=pl.BlockSpec((TC_BLOCK, N_EXP),
                           lambda i: (i + SPLIT // TC_BLOCK, 0)),
    compiler_params=pltpu.CompilerParams(
        dimension_semantics=("arbitrary",)),
)


def kernel(input):
    assert input.shape[0] == NUM_TOKENS
    full = _tc_gate()
    sc = _sc_gate_fn()()
    return lax.dynamic_update_slice(full, sc, (0, 0))


# speedup vs baseline: 1.0242x; 1.0231x over previous
import functools

import jax
import jax.numpy as jnp
from jax import lax
from jax.experimental import pallas as pl
from jax.experimental.pallas import tpu as pltpu
from jax.experimental.pallas import tpu_sc as plsc

NUM_TOKENS = 32768
N_EXP = 64
NC = 2
NS = 16
NW = NC * NS

SPLIT = 7168
TC_BLOCK = 1024

_KS0 = 0
_KS1 = 42
_KS2 = _KS1 ^ 0x1BD11BDA
_ROT_A = (13, 15, 26, 6)
_ROT_B = (17, 29, 16, 24)


def _rotl(x, r):
    return lax.shift_left(x, jnp.int32(r)) | lax.shift_right_logical(
        x, jnp.int32(32 - r))


def _threefry_bits23(j):
    x1 = j + jnp.int32(_KS1)
    x0 = x1
    x1 = _rotl(x1, _ROT_A[0]) ^ x0
    ks = (jnp.int32(_KS0), jnp.int32(_KS1), jnp.int32(_KS2))
    first = True
    for g in range(5):
        rots = _ROT_A if g % 2 == 0 else _ROT_B
        for r in (rots[1:] if first else rots):
            x0 = x0 + x1
            x1 = _rotl(x1, r) ^ x0
        first = False
        x0 = x0 + ks[(g + 1) % 3]
        x1 = x1 + ks[(g + 2) % 3] + jnp.int32(g + 1)
    return lax.shift_right_logical(x0 ^ x1, jnp.int32(9))


SC_ROWS = SPLIT
SC_ROWS_PER_W = SC_ROWS // NW
SC_GROUPS = SC_ROWS_PER_W // 16
_UNROLL = 8


def _sc_body(out_hbm, buf, bits):
    wid = lax.axis_index("s") * NC + lax.axis_index("c")
    row0 = wid * SC_ROWS_PER_W
    lane = lax.iota(jnp.int32, 16)
    lane64 = lane * jnp.int32(N_EXP)
    neg1 = jnp.full((16,), -1, jnp.int32)

    def group(g, carry):
        jbase = (row0 + 16 * g) * N_EXP

        def pass1(i, t):
            t63, t62, t61 = t
            e0 = i * _UNROLL
            bs = [_threefry_bits23(lane64 + (jbase + e0 + k))
                  for k in range(_UNROLL)]
            for k, b in enumerate(bs):
                bits[pl.ds((e0 + k) * 16, 16)] = b
                hi = jnp.maximum(t63, b)
                c1 = jnp.minimum(t63, b)
                mid = jnp.maximum(t62, c1)
                c2 = jnp.minimum(t62, c1)
                lo = jnp.maximum(t61, c2)
                t63, t62, t61 = hi, mid, lo
            return t63, t62, t61

        _, t62, t61 = lax.fori_loop(0, N_EXP // _UNROLL, pass1,
                                    (neg1, neg1, neg1))
        strict = t62 > t61

        zero32 = jnp.zeros((16,), jnp.int32)
        w0, w1 = zero32, zero32
        for e in range(N_EXP):
            b = bits[pl.ds(e * 16, 16)]
            sel = (b > t62) | ((b == t62) & strict)
            bitval = (1 << (e % 32)) - (1 << 32 if e % 32 == 31 else 0)
            bit = jnp.where(sel, jnp.int32(bitval), jnp.int32(0))
            if e < 32:
                w0 = w0 | bit
            else:
                w1 = w1 | bit
        buf[pl.ds(g * 16, 16)] = w0
        buf[pl.ds(SC_ROWS_PER_W + g * 16, 16)] = w1
        return carry

    lax.fori_loop(0, SC_GROUPS, group, 0)
    pltpu.sync_copy(buf.at[pl.ds(0, SC_ROWS_PER_W)],
                    out_hbm.at[pl.ds(row0, SC_ROWS_PER_W)])
    pltpu.sync_copy(buf.at[pl.ds(SC_ROWS_PER_W, SC_ROWS_PER_W)],
                    out_hbm.at[pl.ds(SC_ROWS + row0, SC_ROWS_PER_W)])


@functools.cache
def _sc_gate_fn():
    return pl.kernel(
        _sc_body,
        out_type=jax.ShapeDtypeStruct((2 * SC_ROWS,), jnp.int32),
        mesh=plsc.VectorSubcoreMesh(core_axis_name="c", subcore_axis_name="s"),
        scratch_types=[
            pltpu.VMEM((2 * SC_ROWS_PER_W,), jnp.int32),
            pltpu.VMEM((N_EXP * 16,), jnp.int32),
        ],
        compiler_params=pltpu.CompilerParams(needs_layout_passes=False),
    )


TC_ROWS = NUM_TOKENS - SPLIT


def _tc_kernel(out_ref):
    i = pl.program_id(0)
    row0 = jnp.int32(SPLIT) + i * jnp.int32(TC_BLOCK)
    j = (row0 * jnp.int32(N_EXP)
         + lax.broadcasted_iota(jnp.int32, (N_EXP, TC_BLOCK), 1)
         * jnp.int32(N_EXP)
         + lax.broadcasted_iota(jnp.int32, (N_EXP, TC_BLOCK), 0))
    b = _threefry_bits23(j)
    m1 = jnp.max(b, axis=0, keepdims=True)
    e1 = b == m1
    z = jnp.where(e1, jnp.int32(-1), b)
    m2 = jnp.max(z, axis=0, keepdims=True)
    e2 = z == m2
    y = jnp.where(e2, jnp.int32(-1), z)
    m3 = jnp.max(y, axis=0, keepdims=True)
    c1 = jnp.sum(e1.astype(jnp.int32), axis=0, keepdims=True)
    c2 = jnp.sum(e2.astype(jnp.int32), axis=0, keepdims=True)
    t62 = jnp.where(c1 >= 2, m1, m2)
    t61 = jnp.where(c1 >= 3, m1, jnp.where(c1 + c2 >= 3, m2, m3))
    sel = (b > t62) | ((b == t62) & (t62 > t61))
    mask = jnp.where(sel, jnp.float32(1.0), jnp.float32(0.0))
    out_ref[...] = mask.T


_tc_gate = pl.pallas_call(
    _tc_kernel,
    out_shape=jax.ShapeDtypeStruct((NUM_TOKENS, N_EXP), jnp.float32),
    grid=(TC_ROWS // TC_BLOCK,),
    out_specs=pl.BlockSpec((TC_BLOCK, N_EXP),
                           lambda i: (i + SPLIT // TC_BLOCK, 0)),
    compiler_params=pltpu.CompilerParams(
        dimension_semantics=("arbitrary",)),
)


def kernel(input):
    assert input.shape[0] == NUM_TOKENS
    full = _tc_gate()
    sc_bits = _sc_gate_fn()()
    w = sc_bits.reshape(2, SC_ROWS).T
    shifts = jnp.arange(32, dtype=jnp.int32)
    sc_mask = (((w[:, :, None] >> shifts[None, None, :]) & 1)
               .reshape(SC_ROWS, N_EXP).astype(jnp.float32))
    return lax.dynamic_update_slice(full, sc_mask, (0, 0))
